# baseline (speedup 1.0000x reference)
import functools

import jax
import jax.numpy as jnp
import numpy as np
from jax import lax
from jax.experimental import pallas as pl
from jax.experimental.pallas import tpu as pltpu
from jax.experimental.pallas import tpu_sc as plsc

N = 10000
E = 320000
T = 500000
C = 128
F = 16
CUTOFF = 5.0

_NC = 2
_NS = 16
_NW = _NC * _NS
_W = 128

_TP = 524288


def _silu(x):
    return x * jax.nn.sigmoid(x)


def _pre_kernel(x_ref, w_ref, o_ref):
    o_ref[...] = jnp.dot(x_ref[...], w_ref[...], preferred_element_type=jnp.float32)


def _matmul_pre(features, W_pre):
    blk = 2000
    return pl.pallas_call(
        _pre_kernel,
        grid=(N // blk,),
        in_specs=[
            pl.BlockSpec((blk, C), lambda i: (i, 0)),
            pl.BlockSpec((C, C), lambda i: (0, 0)),
        ],
        out_specs=pl.BlockSpec((blk, C), lambda i: (i, 0)),
        out_shape=jax.ShapeDtypeStruct((N, C), jnp.float32),
    )(features, W_pre)


def _tb_kernel(d_ref, m1_ref, m2_ref, g1_ref, g2_ref, o_ref):
    nrows, nlanes = d_ref.shape
    ne = nrows * nlanes
    width = CUTOFF / F
    dT = d_ref[...].reshape(1, ne)
    centers = lax.broadcasted_iota(jnp.int32, (F, ne), 0).astype(jnp.float32) * (
        CUTOFF / (F - 1))
    rbT = jnp.exp(-((dT - centers) ** 2) * (1.0 / (width * width)))
    cut = 0.5 * (1.0 + jnp.cos(np.pi * dT / CUTOFF)) * (dT < CUTOFF).astype(jnp.float32)
    rbT = rbT * cut
    cdim = (((0,), (0,)), ((), ()))
    h1m = lax.dot_general(m1_ref[...], rbT, cdim, preferred_element_type=jnp.float32)
    h1g = lax.dot_general(g1_ref[...], rbT, cdim, preferred_element_type=jnp.float32)
    mlp = lax.dot_general(_silu(h1m), m2_ref[...], cdim, preferred_element_type=jnp.float32)
    gate = jax.nn.sigmoid(
        lax.dot_general(_silu(h1g), g2_ref[...], cdim, preferred_element_type=jnp.float32))
    o_ref[...] = mlp * gate


def _two_body_weights(d_rows, m1, m2, g1, g2):
    ep = d_rows.shape[0] * 128
    rows = 64
    return pl.pallas_call(
        _tb_kernel,
        grid=(d_rows.shape[0] // rows,),
        in_specs=[
            pl.BlockSpec((rows, 128), lambda i: (i, 0)),
            pl.BlockSpec((F, F), lambda i: (0, 0)),
            pl.BlockSpec((F, C), lambda i: (0, 0)),
            pl.BlockSpec((F, F), lambda i: (0, 0)),
            pl.BlockSpec((F, C), lambda i: (0, 0)),
        ],
        out_specs=pl.BlockSpec((rows * 128, C), lambda i: (i, 0)),
        out_shape=jax.ShapeDtypeStruct((ep, C), jnp.float32),
    )(d_rows, m1, m2, g1, g2)


def _three_kernel(rij_ref, rik_ref, ang_ref, m1_ref, m2_ref, g1_ref, g2_ref, o_ref):
    nrows, nlanes = rij_ref.shape
    ne = nrows * nlanes
    rij = rij_ref[...].reshape(1, ne)
    rik = rik_ref[...].reshape(1, ne)
    ca = jnp.cos(ang_ref[...]).reshape(1, ne)
    featT = jnp.concatenate([rij, rik, ca], axis=0)
    cdim = (((0,), (0,)), ((), ()))
    m1 = m1_ref[...]
    g1 = g1_ref[...]
    h1m = lax.dot_general(m1[:3, :], featT, cdim, preferred_element_type=jnp.float32)
    h1g = lax.dot_general(g1[:3, :], featT, cdim, preferred_element_type=jnp.float32)
    mlp = lax.dot_general(_silu(h1m), m2_ref[...], cdim, preferred_element_type=jnp.float32)
    gate = jax.nn.sigmoid(
        lax.dot_general(_silu(h1g), g2_ref[...], cdim, preferred_element_type=jnp.float32))
    o_ref[...] = mlp * gate


def _three_body_weights(rij_rows, rik_rows, ang_rows, m1p, m2, g1p, g2):
    rows = 32
    return pl.pallas_call(
        _three_kernel,
        grid=(rij_rows.shape[0] // rows,),
        in_specs=[
            pl.BlockSpec((rows, 128), lambda i: (i, 0)),
            pl.BlockSpec((rows, 128), lambda i: (i, 0)),
            pl.BlockSpec((rows, 128), lambda i: (i, 0)),
            pl.BlockSpec((8, 64), lambda i: (0, 0)),
            pl.BlockSpec((64, C), lambda i: (0, 0)),
            pl.BlockSpec((8, 64), lambda i: (0, 0)),
            pl.BlockSpec((64, C), lambda i: (0, 0)),
        ],
        out_specs=pl.BlockSpec((rows * 128, C), lambda i: (i, 0)),
        out_shape=jax.ShapeDtypeStruct((_TP, C), jnp.float32),
    )(rij_rows, rik_rows, ang_rows, m1p, m2, g1p, g2)


def _em_kernel(h_ref, p_ref, o_ref):
    o_ref[...] = h_ref[...] * (p_ref[0] + p_ref[1])


def _em_mul(h, aggp):
    blk = 2000
    return pl.pallas_call(
        _em_kernel,
        grid=(N // blk,),
        in_specs=[
            pl.BlockSpec((blk, C), lambda i: (i, 0)),
            pl.BlockSpec((_NC, blk, C), lambda i: (0, i, 0)),
        ],
        out_specs=pl.BlockSpec((blk, C), lambda i: (i, 0)),
        out_shape=jax.ShapeDtypeStruct((N, C), jnp.float32),
    )(h, aggp)


def _post_kernel(hp_ref, w_ref, o_ref):
    x = hp_ref[0] + hp_ref[1]
    o_ref[...] = jnp.dot(x, w_ref[...], preferred_element_type=jnp.float32)


def _matmul_post(hp, W_post):
    blk = 2000
    return pl.pallas_call(
        _post_kernel,
        grid=(N // blk,),
        in_specs=[
            pl.BlockSpec((_NC, blk, C), lambda i: (0, i, 0)),
            pl.BlockSpec((C, C), lambda i: (0, 0)),
        ],
        out_specs=pl.BlockSpec((blk, C), lambda i: (i, 0)),
        out_shape=jax.ShapeDtypeStruct((N, C), jnp.float32),
    )(hp, W_post)


_RPS = 624
_RTAIL = N - _RPS * _NS


def _striped_rows_copy(src, dst, s):
    pltpu.sync_copy(src.at[pl.ds(s * _RPS, _RPS)], dst.at[pl.ds(s * _RPS, _RPS)])

    @pl.when(s == _NS - 1)
    def _():
        pltpu.sync_copy(src.at[pl.ds(_RPS * _NS, _RTAIL)],
                        dst.at[pl.ds(_RPS * _NS, _RTAIL)])

def _sc_scatter_triplets(three_w, t1p3, zeros_nc):
    wpw = _TP // _W // _NW
    nsb = 8
    sw = wpw // nsb

    mesh = plsc.VectorSubcoreMesh(core_axis_name="c", subcore_axis_name="s")

    @functools.partial(
        pl.kernel,
        out_type=jax.ShapeDtypeStruct((_NC, N, C), jnp.float32),
        mesh=mesh,
        scratch_types=[
            pltpu.VMEM((sw, _W), jnp.int32),
            pltpu.VMEM((_W, C), jnp.float32),
            pltpu.VMEM((_W, C), jnp.float32),
            pltpu.VMEM_SHARED((N, C), jnp.float32),
            pltpu.SemaphoreType.DMA,
            pltpu.SemaphoreType.DMA,
            pltpu.SemaphoreType.DMA,
            pltpu.SemaphoreType.DMA,
        ],
    )
    def k(w_hbm, t1_hbm, z_hbm, out_hbm, idxblk, r_a, r_b, acc_sh,
          ina, inb, sca, scb):
        c = lax.axis_index("c")
        s = lax.axis_index("s")
        wid = s * _NC + c
        row0 = wid * wpw * _W
        _striped_rows_copy(z_hbm, acc_sh, s)
        plsc.subcore_barrier()

        def rsrc(w):
            return w_hbm.at[pl.ds(row0 + w * _W, _W)]

        pltpu.async_copy(rsrc(0), r_a, ina)
        pltpu.async_copy(rsrc(1), r_b, inb)

        def sb_body(sb, carry):
            pltpu.sync_copy(t1_hbm.at[wid, pl.ds(sb * sw, sw)], idxblk)

            def pair_body(j, cc):
                w0 = sb * sw + 2 * j
                pltpu.make_async_copy(rsrc(w0), r_a, ina).wait()
                pltpu.async_copy(r_a, acc_sh.at[idxblk.at[2 * j]], sca, add=True)
                pltpu.make_async_copy(rsrc(w0 + 1), r_b, inb).wait()
                pltpu.async_copy(r_b, acc_sh.at[idxblk.at[2 * j + 1]], scb, add=True)
                pltpu.make_async_copy(r_a, acc_sh.at[idxblk.at[2 * j]], sca).wait()

                @pl.when(w0 + 2 < wpw)
                def _():
                    pltpu.async_copy(rsrc(w0 + 2), r_a, ina)

                pltpu.make_async_copy(r_b, acc_sh.at[idxblk.at[2 * j + 1]], scb).wait()

                @pl.when(w0 + 3 < wpw)
                def _():
                    pltpu.async_copy(rsrc(w0 + 3), r_b, inb)

                return cc

            lax.fori_loop(0, sw // 2, pair_body, 0)
            return carry

        lax.fori_loop(0, nsb, sb_body, 0)

        plsc.subcore_barrier()
        _striped_rows_copy(acc_sh, out_hbm.at[c], s)

    return k(three_w, t1p3, zeros_nc)


_WB = 64


def _sc_edges(tb_w, idx0p3, idx1p3, h, em, em_idx, zeros_nc):
    nwin = tb_w.shape[0] // _WB
    wpw = nwin // _NW
    nsb = 10
    sw = wpw // nsb
    nem = em.shape[0] // _WB // _NW

    mesh = plsc.VectorSubcoreMesh(core_axis_name="c", subcore_axis_name="s")

    @functools.partial(
        pl.kernel,
        out_type=jax.ShapeDtypeStruct((_NC, N, C), jnp.float32),
        mesh=mesh,
        scratch_types=[
            pltpu.VMEM((sw, _WB), jnp.int32),
            pltpu.VMEM((sw, _WB), jnp.int32),
            pltpu.VMEM((_WB, C), jnp.float32),
            pltpu.VMEM((_WB, C), jnp.float32),
            pltpu.VMEM((_WB, C), jnp.float32),
            pltpu.VMEM((_WB, C), jnp.float32),
            pltpu.VMEM_SHARED((N, C), jnp.float32),
            pltpu.SemaphoreType.DMA,
            pltpu.SemaphoreType.DMA,
            pltpu.SemaphoreType.DMA,
            pltpu.SemaphoreType.DMA,
            pltpu.SemaphoreType.DMA,
            pltpu.SemaphoreType.DMA,
        ],
    )
    def k(w_hbm, i0_hbm, i1_hbm, h_hbm, em_hbm, emi_hbm, z_hbm, out_hbm,
          i0blk, i1blk, w_a, w_b, h_a, h_b, acc_sh,
          ga, gb, ina, inb, sca, scb):
        c = lax.axis_index("c")
        s = lax.axis_index("s")
        wid = s * _NC + c
        row0 = wid * wpw * _WB
        _striped_rows_copy(z_hbm, acc_sh, s)
        plsc.subcore_barrier()

        def wsrc(w):
            return w_hbm.at[pl.ds(row0 + w * _WB, _WB)]

        def mul(dst, srcr):
            def mbody(i, cc):
                for j in range(C // 16):
                    sl = pl.ds(j * 16, 16)
                    dst[i, sl] = dst[i, sl] * srcr[i, sl]
                return cc

            lax.fori_loop(0, _WB, mbody, 0)

        def sb_body(sb, carry):
            pltpu.sync_copy(i0_hbm.at[wid, pl.ds(sb * sw, sw)], i0blk)
            pltpu.sync_copy(i1_hbm.at[wid, pl.ds(sb * sw, sw)], i1blk)
            w00 = sb * sw
            pltpu.async_copy(h_hbm.at[i1blk.at[0]], h_a, ga)
            pltpu.async_copy(wsrc(w00), w_a, ina)
            pltpu.async_copy(h_hbm.at[i1blk.at[1]], h_b, gb)
            pltpu.async_copy(wsrc(w00 + 1), w_b, inb)

            def pair_body(j, cc):
                w0 = w00 + 2 * j
                pltpu.make_async_copy(h_hbm.at[i1blk.at[2 * j]], h_a, ga).wait()
                pltpu.make_async_copy(wsrc(w0), w_a, ina).wait()
                mul(w_a, h_a)
                pltpu.async_copy(w_a, acc_sh.at[i0blk.at[2 * j]], sca, add=True)
                pltpu.make_async_copy(h_hbm.at[i1blk.at[2 * j + 1]], h_b, gb).wait()
                pltpu.make_async_copy(wsrc(w0 + 1), w_b, inb).wait()
                mul(w_b, h_b)
                pltpu.async_copy(w_b, acc_sh.at[i0blk.at[2 * j + 1]], scb, add=True)
                pltpu.make_async_copy(w_a, acc_sh.at[i0blk.at[2 * j]], sca).wait()

                @pl.when(j < sw // 2 - 1)
                def _():
                    pltpu.async_copy(h_hbm.at[i1blk.at[2 * j + 2]], h_a, ga)
                    pltpu.async_copy(wsrc(w0 + 2), w_a, ina)

                pltpu.make_async_copy(w_b, acc_sh.at[i0blk.at[2 * j + 1]], scb).wait()

                @pl.when(j < sw // 2 - 1)
                def _():
                    pltpu.async_copy(h_hbm.at[i1blk.at[2 * j + 3]], h_b, gb)
                    pltpu.async_copy(wsrc(w0 + 3), w_b, inb)

                return cc

            lax.fori_loop(0, sw // 2, pair_body, 0)
            return carry

        lax.fori_loop(0, nsb, sb_body, 0)

        def embody(k2, carry):
            base = (wid * nem + k2) * _WB
            pltpu.sync_copy(emi_hbm.at[pl.ds(base, _WB)], i0blk.at[0])
            pltpu.sync_copy(em_hbm.at[pl.ds(base, _WB)], w_a)
            pltpu.sync_copy(w_a, acc_sh.at[i0blk.at[0]], add=True)
            return carry

        lax.fori_loop(0, nem, embody, 0)

        plsc.subcore_barrier()
        _striped_rows_copy(acc_sh, out_hbm.at[c], s)

    return k(tb_w, idx0p3, idx1p3, h, em, em_idx, zeros_nc)


def kernel(features, neighbour_distances, edge_index, triplet_idxs, angles,
           r_ij, r_ik, W_pre, tb_m1, tb_m2, tb_g1, tb_g2, three_m1, three_m2,
           three_g1, three_g2, W_post):
    ep = 327680
    idx0 = edge_index[0].astype(jnp.int32)
    idx0p = jnp.pad(idx0, (0, ep - E))
    idx1p = jnp.pad(edge_index[1].astype(jnp.int32), (0, ep - E))
    d_rows = jnp.pad(neighbour_distances, (0, ep - E),
                     constant_values=np.float32(2 * CUTOFF)).reshape(ep // 128, 128)
    pad = _TP - T
    t1p = jnp.pad(triplet_idxs[:, 1].astype(jnp.int32), (0, pad))
    rij_rows = jnp.pad(r_ij, (0, pad)).reshape(_TP // 128, 128)
    rik_rows = jnp.pad(r_ik, (0, pad)).reshape(_TP // 128, 128)
    ang_rows = jnp.pad(angles, (0, pad),
                       constant_values=np.float32(np.pi / 2)).reshape(_TP // 128, 128)
    m1p = jnp.zeros((8, 64), jnp.float32).at[:3].set(three_m1)
    g1p = jnp.zeros((8, 64), jnp.float32).at[:3].set(three_g1)
    zeros_nc = jnp.zeros((N, C), jnp.float32)

    h = _matmul_pre(features, W_pre)
    tb_w = _two_body_weights(d_rows, tb_m1, tb_m2, tb_g1, tb_g2)
    three_w = _three_body_weights(rij_rows, rik_rows, ang_rows, m1p, three_m2,
                                  g1p, three_g2)
    aggp = _sc_scatter_triplets(three_w, t1p.reshape(_NW, 128, _W), zeros_nc)
    em = jnp.pad(_em_mul(h, aggp), ((0, 240), (0, 0)))
    emi = jnp.pad(idx0[:N], (0, 240))
    hp = _sc_edges(tb_w, idx0p.reshape(_NW, 160, _WB),
                   idx1p.reshape(_NW, 160, _WB), h, em, emi, zeros_nc)
    return _matmul_post(hp, W_post)


# speedup vs baseline: 1.0256x; 1.0256x over previous
import functools

import jax
import jax.numpy as jnp
import numpy as np
from jax import lax
from jax.experimental import pallas as pl
from jax.experimental.pallas import tpu as pltpu
from jax.experimental.pallas import tpu_sc as plsc

N = 10000
E = 320000
T = 500000
C = 128
F = 16
CUTOFF = 5.0

_NC = 2
_NS = 16
_NW = _NC * _NS
_W = 128

_TP = 524288


def _silu(x):
    return x * jax.nn.sigmoid(x)


def _pre_kernel(x_ref, w_ref, o_ref):
    o_ref[...] = jnp.dot(x_ref[...], w_ref[...], preferred_element_type=jnp.float32)


def _matmul_pre(features, W_pre):
    blk = 2000
    return pl.pallas_call(
        _pre_kernel,
        grid=(N // blk,),
        in_specs=[
            pl.BlockSpec((blk, C), lambda i: (i, 0)),
            pl.BlockSpec((C, C), lambda i: (0, 0)),
        ],
        out_specs=pl.BlockSpec((blk, C), lambda i: (i, 0)),
        out_shape=jax.ShapeDtypeStruct((N, C), jnp.float32),
    )(features, W_pre)


def _tb_kernel(d_ref, m1_ref, m2_ref, g1_ref, g2_ref, o_ref):
    nrows, nlanes = d_ref.shape
    ne = nrows * nlanes
    width = CUTOFF / F
    dT = d_ref[...].reshape(1, ne)
    centers = lax.broadcasted_iota(jnp.int32, (F, ne), 0).astype(jnp.float32) * (
        CUTOFF / (F - 1))
    rbT = jnp.exp(-((dT - centers) ** 2) * (1.0 / (width * width)))
    cut = 0.5 * (1.0 + jnp.cos(np.pi * dT / CUTOFF)) * (dT < CUTOFF).astype(jnp.float32)
    rbT = rbT * cut
    cdim = (((0,), (0,)), ((), ()))
    h1m = lax.dot_general(m1_ref[...], rbT, cdim, preferred_element_type=jnp.float32)
    h1g = lax.dot_general(g1_ref[...], rbT, cdim, preferred_element_type=jnp.float32)
    mlp = lax.dot_general(_silu(h1m), m2_ref[...], cdim, preferred_element_type=jnp.float32)
    gate = jax.nn.sigmoid(
        lax.dot_general(_silu(h1g), g2_ref[...], cdim, preferred_element_type=jnp.float32))
    o_ref[...] = mlp * gate


def _two_body_weights(d_rows, m1, m2, g1, g2):
    ep = d_rows.shape[0] * 128
    rows = 64
    return pl.pallas_call(
        _tb_kernel,
        grid=(d_rows.shape[0] // rows,),
        in_specs=[
            pl.BlockSpec((rows, 128), lambda i: (i, 0)),
            pl.BlockSpec((F, F), lambda i: (0, 0)),
            pl.BlockSpec((F, C), lambda i: (0, 0)),
            pl.BlockSpec((F, F), lambda i: (0, 0)),
            pl.BlockSpec((F, C), lambda i: (0, 0)),
        ],
        out_specs=pl.BlockSpec((rows * 128, C), lambda i: (i, 0)),
        out_shape=jax.ShapeDtypeStruct((ep, C), jnp.float32),
    )(d_rows, m1, m2, g1, g2)


def _three_kernel(rij_ref, rik_ref, ang_ref, m1_ref, m2_ref, g1_ref, g2_ref, o_ref):
    nrows, nlanes = rij_ref.shape
    ne = nrows * nlanes
    rij = rij_ref[...].reshape(1, ne)
    rik = rik_ref[...].reshape(1, ne)
    ca = jnp.cos(ang_ref[...]).reshape(1, ne)
    featT = jnp.concatenate([rij, rik, ca], axis=0)
    cdim = (((0,), (0,)), ((), ()))
    m1 = m1_ref[...]
    g1 = g1_ref[...]
    h1m = lax.dot_general(m1[:3, :], featT, cdim, preferred_element_type=jnp.float32)
    h1g = lax.dot_general(g1[:3, :], featT, cdim, preferred_element_type=jnp.float32)
    mlp = lax.dot_general(_silu(h1m).astype(jnp.bfloat16),
                          m2_ref[...].astype(jnp.bfloat16), cdim,
                          preferred_element_type=jnp.float32)
    gate = jax.nn.sigmoid(
        lax.dot_general(_silu(h1g).astype(jnp.bfloat16),
                        g2_ref[...].astype(jnp.bfloat16), cdim,
                        preferred_element_type=jnp.float32))
    o_ref[...] = mlp * gate


def _three_body_weights(rij_rows, rik_rows, ang_rows, m1p, m2, g1p, g2):
    rows = 32
    return pl.pallas_call(
        _three_kernel,
        grid=(rij_rows.shape[0] // rows,),
        in_specs=[
            pl.BlockSpec((rows, 128), lambda i: (i, 0)),
            pl.BlockSpec((rows, 128), lambda i: (i, 0)),
            pl.BlockSpec((rows, 128), lambda i: (i, 0)),
            pl.BlockSpec((8, 64), lambda i: (0, 0)),
            pl.BlockSpec((64, C), lambda i: (0, 0)),
            pl.BlockSpec((8, 64), lambda i: (0, 0)),
            pl.BlockSpec((64, C), lambda i: (0, 0)),
        ],
        out_specs=pl.BlockSpec((rows * 128, C), lambda i: (i, 0)),
        out_shape=jax.ShapeDtypeStruct((_TP, C), jnp.float32),
    )(rij_rows, rik_rows, ang_rows, m1p, m2, g1p, g2)


def _em_kernel(h_ref, p_ref, o_ref):
    o_ref[...] = h_ref[...] * (p_ref[0] + p_ref[1])


def _em_mul(h, aggp):
    blk = 2000
    return pl.pallas_call(
        _em_kernel,
        grid=(N // blk,),
        in_specs=[
            pl.BlockSpec((blk, C), lambda i: (i, 0)),
            pl.BlockSpec((_NC, blk, C), lambda i: (0, i, 0)),
        ],
        out_specs=pl.BlockSpec((blk, C), lambda i: (i, 0)),
        out_shape=jax.ShapeDtypeStruct((N, C), jnp.float32),
    )(h, aggp)


def _post_kernel(hp_ref, w_ref, o_ref):
    x = hp_ref[0] + hp_ref[1]
    o_ref[...] = jnp.dot(x, w_ref[...], preferred_element_type=jnp.float32)


def _matmul_post(hp, W_post):
    blk = 2000
    return pl.pallas_call(
        _post_kernel,
        grid=(N // blk,),
        in_specs=[
            pl.BlockSpec((_NC, blk, C), lambda i: (0, i, 0)),
            pl.BlockSpec((C, C), lambda i: (0, 0)),
        ],
        out_specs=pl.BlockSpec((blk, C), lambda i: (i, 0)),
        out_shape=jax.ShapeDtypeStruct((N, C), jnp.float32),
    )(hp, W_post)


_RPS = 624
_RTAIL = N - _RPS * _NS


def _striped_rows_copy(src, dst, s):
    pltpu.sync_copy(src.at[pl.ds(s * _RPS, _RPS)], dst.at[pl.ds(s * _RPS, _RPS)])

    @pl.when(s == _NS - 1)
    def _():
        pltpu.sync_copy(src.at[pl.ds(_RPS * _NS, _RTAIL)],
                        dst.at[pl.ds(_RPS * _NS, _RTAIL)])

def _sc_scatter_triplets(three_w, t1p3, zeros_nc):
    wpw = _TP // _W // _NW
    nsb = 8
    sw = wpw // nsb

    mesh = plsc.VectorSubcoreMesh(core_axis_name="c", subcore_axis_name="s")

    @functools.partial(
        pl.kernel,
        out_type=jax.ShapeDtypeStruct((_NC, N, C), jnp.float32),
        mesh=mesh,
        scratch_types=[
            pltpu.VMEM((sw, _W), jnp.int32),
            pltpu.VMEM((_W, C), jnp.float32),
            pltpu.VMEM((_W, C), jnp.float32),
            pltpu.VMEM_SHARED((N, C), jnp.float32),
            pltpu.SemaphoreType.DMA,
            pltpu.SemaphoreType.DMA,
            pltpu.SemaphoreType.DMA,
            pltpu.SemaphoreType.DMA,
        ],
    )
    def k(w_hbm, t1_hbm, z_hbm, out_hbm, idxblk, r_a, r_b, acc_sh,
          ina, inb, sca, scb):
        c = lax.axis_index("c")
        s = lax.axis_index("s")
        wid = s * _NC + c
        row0 = wid * wpw * _W
        _striped_rows_copy(z_hbm, acc_sh, s)
        plsc.subcore_barrier()

        def rsrc(w):
            return w_hbm.at[pl.ds(row0 + w * _W, _W)]

        pltpu.async_copy(rsrc(0), r_a, ina)
        pltpu.async_copy(rsrc(1), r_b, inb)

        def sb_body(sb, carry):
            pltpu.sync_copy(t1_hbm.at[wid, pl.ds(sb * sw, sw)], idxblk)

            def pair_body(j, cc):
                w0 = sb * sw + 2 * j
                pltpu.make_async_copy(rsrc(w0), r_a, ina).wait()
                pltpu.async_copy(r_a, acc_sh.at[idxblk.at[2 * j]], sca, add=True)
                pltpu.make_async_copy(rsrc(w0 + 1), r_b, inb).wait()
                pltpu.async_copy(r_b, acc_sh.at[idxblk.at[2 * j + 1]], scb, add=True)
                pltpu.make_async_copy(r_a, acc_sh.at[idxblk.at[2 * j]], sca).wait()

                @pl.when(w0 + 2 < wpw)
                def _():
                    pltpu.async_copy(rsrc(w0 + 2), r_a, ina)

                pltpu.make_async_copy(r_b, acc_sh.at[idxblk.at[2 * j + 1]], scb).wait()

                @pl.when(w0 + 3 < wpw)
                def _():
                    pltpu.async_copy(rsrc(w0 + 3), r_b, inb)

                return cc

            lax.fori_loop(0, sw // 2, pair_body, 0)
            return carry

        lax.fori_loop(0, nsb, sb_body, 0)

        plsc.subcore_barrier()
        _striped_rows_copy(acc_sh, out_hbm.at[c], s)

    return k(three_w, t1p3, zeros_nc)


_WB = 64


def _sc_edges(tb_w, idx0p3, idx1p3, h, em, em_idx, zeros_nc):
    nwin = tb_w.shape[0] // _WB
    wpw = nwin // _NW
    nsb = 10
    sw = wpw // nsb
    nem = em.shape[0] // _WB // _NW

    mesh = plsc.VectorSubcoreMesh(core_axis_name="c", subcore_axis_name="s")

    @functools.partial(
        pl.kernel,
        out_type=jax.ShapeDtypeStruct((_NC, N, C), jnp.float32),
        mesh=mesh,
        scratch_types=[
            pltpu.VMEM((sw, _WB), jnp.int32),
            pltpu.VMEM((sw, _WB), jnp.int32),
            pltpu.VMEM((_WB, C), jnp.float32),
            pltpu.VMEM((_WB, C), jnp.float32),
            pltpu.VMEM((_WB, C), jnp.float32),
            pltpu.VMEM((_WB, C), jnp.float32),
            pltpu.VMEM_SHARED((N, C), jnp.float32),
            pltpu.SemaphoreType.DMA,
            pltpu.SemaphoreType.DMA,
            pltpu.SemaphoreType.DMA,
            pltpu.SemaphoreType.DMA,
            pltpu.SemaphoreType.DMA,
            pltpu.SemaphoreType.DMA,
        ],
    )
    def k(w_hbm, i0_hbm, i1_hbm, h_hbm, em_hbm, emi_hbm, z_hbm, out_hbm,
          i0blk, i1blk, w_a, w_b, h_a, h_b, acc_sh,
          ga, gb, ina, inb, sca, scb):
        c = lax.axis_index("c")
        s = lax.axis_index("s")
        wid = s * _NC + c
        row0 = wid * wpw * _WB
        _striped_rows_copy(z_hbm, acc_sh, s)
        plsc.subcore_barrier()

        def wsrc(w):
            return w_hbm.at[pl.ds(row0 + w * _WB, _WB)]

        def mul(dst, srcr):
            def mbody(i, cc):
                for j in range(C // 16):
                    sl = pl.ds(j * 16, 16)
                    dst[i, sl] = dst[i, sl] * srcr[i, sl]
                return cc

            lax.fori_loop(0, _WB, mbody, 0)

        def sb_body(sb, carry):
            pltpu.sync_copy(i0_hbm.at[wid, pl.ds(sb * sw, sw)], i0blk)
            pltpu.sync_copy(i1_hbm.at[wid, pl.ds(sb * sw, sw)], i1blk)
            w00 = sb * sw
            pltpu.async_copy(h_hbm.at[i1blk.at[0]], h_a, ga)
            pltpu.async_copy(wsrc(w00), w_a, ina)
            pltpu.async_copy(h_hbm.at[i1blk.at[1]], h_b, gb)
            pltpu.async_copy(wsrc(w00 + 1), w_b, inb)

            def pair_body(j, cc):
                w0 = w00 + 2 * j
                pltpu.make_async_copy(h_hbm.at[i1blk.at[2 * j]], h_a, ga).wait()
                pltpu.make_async_copy(wsrc(w0), w_a, ina).wait()
                mul(w_a, h_a)
                pltpu.async_copy(w_a, acc_sh.at[i0blk.at[2 * j]], sca, add=True)
                pltpu.make_async_copy(h_hbm.at[i1blk.at[2 * j + 1]], h_b, gb).wait()
                pltpu.make_async_copy(wsrc(w0 + 1), w_b, inb).wait()
                mul(w_b, h_b)
                pltpu.async_copy(w_b, acc_sh.at[i0blk.at[2 * j + 1]], scb, add=True)
                pltpu.make_async_copy(w_a, acc_sh.at[i0blk.at[2 * j]], sca).wait()

                @pl.when(j < sw // 2 - 1)
                def _():
                    pltpu.async_copy(h_hbm.at[i1blk.at[2 * j + 2]], h_a, ga)
                    pltpu.async_copy(wsrc(w0 + 2), w_a, ina)

                pltpu.make_async_copy(w_b, acc_sh.at[i0blk.at[2 * j + 1]], scb).wait()

                @pl.when(j < sw // 2 - 1)
                def _():
                    pltpu.async_copy(h_hbm.at[i1blk.at[2 * j + 3]], h_b, gb)
                    pltpu.async_copy(wsrc(w0 + 3), w_b, inb)

                return cc

            lax.fori_loop(0, sw // 2, pair_body, 0)
            return carry

        lax.fori_loop(0, nsb, sb_body, 0)

        def embody(k2, carry):
            base = (wid * nem + k2) * _WB
            pltpu.sync_copy(emi_hbm.at[pl.ds(base, _WB)], i0blk.at[0])
            pltpu.sync_copy(em_hbm.at[pl.ds(base, _WB)], w_a)
            pltpu.sync_copy(w_a, acc_sh.at[i0blk.at[0]], add=True)
            return carry

        lax.fori_loop(0, nem, embody, 0)

        plsc.subcore_barrier()
        _striped_rows_copy(acc_sh, out_hbm.at[c], s)

    return k(tb_w, idx0p3, idx1p3, h, em, em_idx, zeros_nc)


def kernel(features, neighbour_distances, edge_index, triplet_idxs, angles,
           r_ij, r_ik, W_pre, tb_m1, tb_m2, tb_g1, tb_g2, three_m1, three_m2,
           three_g1, three_g2, W_post):
    ep = 327680
    idx0 = edge_index[0].astype(jnp.int32)
    idx0p = jnp.pad(idx0, (0, ep - E))
    idx1p = jnp.pad(edge_index[1].astype(jnp.int32), (0, ep - E))
    d_rows = jnp.pad(neighbour_distances, (0, ep - E),
                     constant_values=np.float32(2 * CUTOFF)).reshape(ep // 128, 128)
    pad = _TP - T
    t1p = jnp.pad(triplet_idxs[:, 1].astype(jnp.int32), (0, pad))
    rij_rows = jnp.pad(r_ij, (0, pad)).reshape(_TP // 128, 128)
    rik_rows = jnp.pad(r_ik, (0, pad)).reshape(_TP // 128, 128)
    ang_rows = jnp.pad(angles, (0, pad),
                       constant_values=np.float32(np.pi / 2)).reshape(_TP // 128, 128)
    m1p = jnp.zeros((8, 64), jnp.float32).at[:3].set(three_m1)
    g1p = jnp.zeros((8, 64), jnp.float32).at[:3].set(three_g1)
    zeros_nc = jnp.zeros((N, C), jnp.float32)

    h = _matmul_pre(features, W_pre)
    tb_w = _two_body_weights(d_rows, tb_m1, tb_m2, tb_g1, tb_g2)
    three_w = _three_body_weights(rij_rows, rik_rows, ang_rows, m1p, three_m2,
                                  g1p, three_g2)
    aggp = _sc_scatter_triplets(three_w, t1p.reshape(_NW, 128, _W), zeros_nc)
    em = jnp.pad(_em_mul(h, aggp), ((0, 240), (0, 0)))
    emi = jnp.pad(idx0[:N], (0, 240))
    hp = _sc_edges(tb_w, idx0p.reshape(_NW, 160, _WB),
                   idx1p.reshape(_NW, 160, _WB), h, em, emi, zeros_nc)
    return _matmul_post(hp, W_post)
